# Initial kernel scaffold; baseline (speedup 1.0000x reference)
import jax
import jax.numpy as jnp
from jax.experimental import pallas as pl


def kernel(target_feature, fc1, fc2, index_target, target_featurememory, target_softmaxFc1memory, target_softmaxFc2memory):
    raise NotImplementedError("write your pallas kernel here")


# speedup vs baseline: 27.2481x; 27.2481x over previous
import functools

import jax
import jax.numpy as jnp
from jax.experimental import pallas as pl
from jax.experimental.pallas import tpu as pltpu

B, D, C, T = 1024, 512, 1000, 50000
TOP = 8
MOM = 0.1
TBLK = 512
NBLK = 98
TPAD = NBLK * TBLK
EPS = 1e-12
NEG = -jnp.inf
NSLOT = 16


def _norm_rows(x):
    n = jnp.sqrt(jnp.sum(x * x, axis=1, keepdims=True))
    return x / jnp.maximum(n, EPS)


def _sim_block(q, mem, maskrow, colvalid):
    rsq = jnp.sum(mem * mem, axis=1)
    rinv = 1.0 / jnp.maximum(jnp.sqrt(rsq), EPS)
    s = jax.lax.dot_general(q, mem, (((1,), (1,)), ((), ())),
                            preferred_element_type=jnp.float32)
    s = s * rinv[None, :] + maskrow
    return jnp.where(colvalid, s, NEG)


def _merge_top9(step, bm, vals_ref, blks_ref):
    slots = jax.lax.broadcasted_iota(jnp.int32, (B, NSLOT), 1)

    @pl.when(step == 0)
    def _():
        vals_ref[...] = jnp.where(slots < 9, NEG, jnp.inf)
        blks_ref[...] = jnp.zeros((B, NSLOT), jnp.int32)

    vals = vals_ref[...]
    cmin = jnp.min(vals, axis=1)
    sel = jnp.where(vals == cmin[:, None], slots, NSLOT + 1)
    p = jnp.min(sel, axis=1)
    hit = (slots == p[:, None]) & (bm > cmin)[:, None]
    vals_ref[...] = jnp.where(hit, bm[:, None], vals)
    blks_ref[...] = jnp.where(hit, step, blks_ref[...])


def _big_kernel(qf_ref, q1_ref, q2_ref, mask_ref, fm_ref, m1_ref, m2_ref,
                sf_ref, s1_ref, s2_ref, bf_ref, b1_ref, b2_ref, rd_ref,
                vf_ref, v1_ref, v2_ref):
    i = pl.program_id(0)
    maskrow = mask_ref[0:1, :]
    col = jax.lax.broadcasted_iota(jnp.int32, (B, TBLK), 1) + i * TBLK
    colvalid = col < T

    sf = _sim_block(qf_ref[...], fm_ref[...], maskrow, colvalid)
    sf_ref[...] = sf
    _merge_top9(i, jnp.max(sf, axis=1), vf_ref, bf_ref)

    m1 = m1_ref[...]
    m2 = m2_ref[...]
    s1 = _sim_block(q1_ref[...], m1, maskrow, colvalid)
    s1_ref[...] = s1
    _merge_top9(i, jnp.max(s1, axis=1), v1_ref, b1_ref)

    s2 = _sim_block(q2_ref[...], m2, maskrow, colvalid)
    s2_ref[...] = s2
    _merge_top9(i, jnp.max(s2, axis=1), v2_ref, b2_ref)

    rd = jnp.sum(jnp.abs(m1 - m2), axis=1)
    rd_ref[...] = jnp.broadcast_to(rd[None, :], (8, TBLK))


@jax.jit
def _big(qf, q1, q2, mask2d, FM, M1, M2):
    grid = (NBLK,)
    res_spec = lambda shape: pl.BlockSpec(shape, lambda i: (0, 0))
    mem_spec = lambda width: pl.BlockSpec((TBLK, width), lambda i: (i, 0))
    s_spec = pl.BlockSpec((B, TBLK), lambda i: (0, i))
    out_shapes = (
        jax.ShapeDtypeStruct((B, TPAD), jnp.float32),
        jax.ShapeDtypeStruct((B, TPAD), jnp.float32),
        jax.ShapeDtypeStruct((B, TPAD), jnp.float32),
        jax.ShapeDtypeStruct((B, NSLOT), jnp.int32),
        jax.ShapeDtypeStruct((B, NSLOT), jnp.int32),
        jax.ShapeDtypeStruct((B, NSLOT), jnp.int32),
        jax.ShapeDtypeStruct((8, TPAD), jnp.float32),
    )
    out_specs = (
        s_spec, s_spec, s_spec,
        res_spec((B, NSLOT)), res_spec((B, NSLOT)), res_spec((B, NSLOT)),
        pl.BlockSpec((8, TBLK), lambda i: (0, i)),
    )
    in_specs = (
        res_spec((B, D)),
        res_spec((B, C)),
        res_spec((B, C)),
        pl.BlockSpec((8, TBLK), lambda i: (0, i)),
        mem_spec(D), mem_spec(C), mem_spec(C),
    )
    scratch = [pltpu.VMEM((B, NSLOT), jnp.float32)] * 3
    return pl.pallas_call(
        _big_kernel,
        grid=grid,
        in_specs=in_specs,
        out_specs=out_specs,
        out_shape=out_shapes,
        scratch_shapes=scratch,
    )(qf, q1, q2, mask2d, FM, M1, M2)


def kernel(target_feature, fc1, fc2, index_target, target_featurememory,
           target_softmaxFc1memory, target_softmaxFc2memory):
    feat = target_feature
    idx = index_target
    FM, M1, M2 = target_featurememory, target_softmaxFc1memory, target_softmaxFc2memory

    s1 = jax.nn.softmax(fc1, axis=1)
    s2 = jax.nn.softmax(fc2, axis=1)
    qf = _norm_rows(feat)
    q1 = _norm_rows(s1)
    q2 = _norm_rows(s2)

    jpos = jnp.arange(B, dtype=jnp.int32)
    w = jnp.full((T,), -1, dtype=jnp.int32).at[idx].max(jpos)
    valid = (w[idx] == jpos)

    gfm, gm1, gm2 = FM[idx], M1[idx], M2[idx]
    nf = (1.0 - MOM) * gfm + MOM * feat
    n1 = (1.0 - MOM) * gm1 + MOM * s1
    n2 = (1.0 - MOM) * gm2 + MOM * s2

    mask1 = jnp.zeros((TPAD,), jnp.float32).at[idx].set(NEG)
    mask2d = jnp.broadcast_to(mask1[None, :], (8, TPAD))

    Sf, S1, S2, bf, b1, b2, rd2d = _big(qf, q1, q2, mask2d, FM, M1, M2)
    rowdiff = rd2d[0, :T]

    def corr(q, nr):
        rn = jnp.sqrt(jnp.sum(nr * nr, axis=1))
        rinv = 1.0 / jnp.maximum(rn, EPS)
        Cm = (q @ nr.T) * rinv[None, :]
        return Cm + jnp.where(valid, 0.0, NEG)[None, :]

    def topk9(S, blks, q, nr):
        blk9 = blks[:, :9]
        Sb = S.reshape(B, NBLK, TBLK)
        cand = jnp.take_along_axis(Sb, blk9[:, :, None], axis=1).reshape(B, 9 * TBLK)
        gidx = (blk9[:, :, None] * TBLK
                + jnp.arange(TBLK, dtype=jnp.int32)[None, None, :]).reshape(B, 9 * TBLK)
        Cm = corr(q, nr)
        allv = jnp.concatenate([cand, Cm], axis=1)
        allg = jnp.concatenate([gidx, jnp.broadcast_to(idx[None, :], (B, B))], axis=1)
        _, p9 = jax.lax.top_k(allv, 9)
        return jnp.take_along_axis(allg, p9, axis=1)

    f_idx = topk9(Sf, bf, qf, nf)[:, 1:9].reshape(-1)
    i1 = topk9(S1, b1, q1, n1)[:, 1:8].reshape(-1)
    i2 = topk9(S2, b2, q2, n2)[:, 1:8].reshape(-1)

    rowdiff_upd = jnp.sum(jnp.abs(n1 - n2), axis=1)
    posf = w[f_idx]
    vals = jnp.where(posf >= 0, rowdiff_upd[jnp.maximum(posf, 0)], rowdiff[f_idx])
    loss_softmax = jnp.sum(vals) / (B * 8 * C)

    def fetch(ii):
        p = w[ii]
        base = FM[ii]
        upd = nf[jnp.maximum(p, 0)]
        return jnp.where((p >= 0)[:, None], upd, base)

    a = _norm_rows(fetch(i1))
    b = _norm_rows(fetch(i2))
    loss_feature = 0.5 - jnp.sum(a * b) / (2.0 * i1.shape[0])

    return (loss_softmax, loss_feature)


# speedup vs baseline: 29.8245x; 1.0946x over previous
import jax
import jax.numpy as jnp
from jax.experimental import pallas as pl
from jax.experimental.pallas import tpu as pltpu

B, D, C, T = 1024, 512, 1000, 50000
TOP = 8
MOM = 0.1
TBLK = 512
NBLK = 98
TPAD = NBLK * TBLK
EPS = 1e-12
NEG = -jnp.inf
NSLOT = 16
NCAND = 9 * TBLK
WTOT = NCAND + B
BIGI = 2 ** 30


def _norm_rows(x):
    n = jnp.sqrt(jnp.sum(x * x, axis=1, keepdims=True))
    return x / jnp.maximum(n, EPS)


def _prep_kernel(feat_ref, fc1_ref, fc2_ref, qf_ref, s1_ref, q1_ref,
                 s2_ref, q2_ref):
    feat = feat_ref[...]
    qf_ref[...] = _norm_rows(feat)

    def soft(x):
        m = jnp.max(x, axis=1, keepdims=True)
        e = jnp.exp(x - m)
        return e / jnp.sum(e, axis=1, keepdims=True)

    s1 = soft(fc1_ref[...])
    s1_ref[...] = s1
    q1_ref[...] = _norm_rows(s1)
    s2 = soft(fc2_ref[...])
    s2_ref[...] = s2
    q2_ref[...] = _norm_rows(s2)


@jax.jit
def _prep(feat, fc1, fc2):
    shp = lambda c: jax.ShapeDtypeStruct((B, c), jnp.float32)
    return pl.pallas_call(
        _prep_kernel,
        out_shape=(shp(D), shp(C), shp(C), shp(C), shp(C)),
    )(feat, fc1, fc2)


def _sim_block(q, mem, maskrow, colvalid):
    rsq = jnp.sum(mem * mem, axis=1)
    rinv = 1.0 / jnp.maximum(jnp.sqrt(rsq), EPS)
    s = jax.lax.dot_general(q, mem, (((1,), (1,)), ((), ())),
                            preferred_element_type=jnp.float32)
    s = s * rinv[None, :] + maskrow
    return jnp.where(colvalid, s, NEG)


def _merge_top9(step, bm, vals_ref, blks_ref):
    slots = jax.lax.broadcasted_iota(jnp.int32, (B, NSLOT), 1)

    @pl.when(step == 0)
    def _():
        vals_ref[...] = jnp.where(slots < 9, NEG, jnp.inf)
        blks_ref[...] = jnp.zeros((B, NSLOT), jnp.int32)

    vals = vals_ref[...]
    cmin = jnp.min(vals, axis=1)
    sel = jnp.where(vals == cmin[:, None], slots, NSLOT + 1)
    p = jnp.min(sel, axis=1)
    hit = (slots == p[:, None]) & (bm > cmin)[:, None]
    vals_ref[...] = jnp.where(hit, bm[:, None], vals)
    blks_ref[...] = jnp.where(hit, step, blks_ref[...])


def _big_kernel(qf_ref, q1_ref, q2_ref, mask_ref, fm_ref, m1_ref, m2_ref,
                sf_ref, s1_ref, s2_ref, bf_ref, b1_ref, b2_ref, rd_ref,
                vf_ref, v1_ref, v2_ref):
    i = pl.program_id(0)
    maskrow = mask_ref[0:1, :]
    col = jax.lax.broadcasted_iota(jnp.int32, (B, TBLK), 1) + i * TBLK
    colvalid = col < T

    sf = _sim_block(qf_ref[...], fm_ref[...], maskrow, colvalid)
    sf_ref[...] = sf
    _merge_top9(i, jnp.max(sf, axis=1), vf_ref, bf_ref)

    m1 = m1_ref[...]
    m2 = m2_ref[...]
    s1 = _sim_block(q1_ref[...], m1, maskrow, colvalid)
    s1_ref[...] = s1
    _merge_top9(i, jnp.max(s1, axis=1), v1_ref, b1_ref)

    s2 = _sim_block(q2_ref[...], m2, maskrow, colvalid)
    s2_ref[...] = s2
    _merge_top9(i, jnp.max(s2, axis=1), v2_ref, b2_ref)

    rd = jnp.sum(jnp.abs(m1 - m2), axis=1)
    rd_ref[...] = jnp.broadcast_to(rd[None, :], (8, TBLK))


@jax.jit
def _big(qf, q1, q2, mask2d, FM, M1, M2):
    res_spec = lambda shape: pl.BlockSpec(shape, lambda i: (0, 0))
    mem_spec = lambda width: pl.BlockSpec((TBLK, width), lambda i: (i, 0))
    s_spec = pl.BlockSpec((B, TBLK), lambda i: (0, i))
    out_shapes = (
        jax.ShapeDtypeStruct((B, TPAD), jnp.float32),
        jax.ShapeDtypeStruct((B, TPAD), jnp.float32),
        jax.ShapeDtypeStruct((B, TPAD), jnp.float32),
        jax.ShapeDtypeStruct((B, NSLOT), jnp.int32),
        jax.ShapeDtypeStruct((B, NSLOT), jnp.int32),
        jax.ShapeDtypeStruct((B, NSLOT), jnp.int32),
        jax.ShapeDtypeStruct((8, TPAD), jnp.float32),
    )
    out_specs = (
        s_spec, s_spec, s_spec,
        res_spec((B, NSLOT)), res_spec((B, NSLOT)), res_spec((B, NSLOT)),
        pl.BlockSpec((8, TBLK), lambda i: (0, i)),
    )
    in_specs = (
        res_spec((B, D)),
        res_spec((B, C)),
        res_spec((B, C)),
        pl.BlockSpec((8, TBLK), lambda i: (0, i)),
        mem_spec(D), mem_spec(C), mem_spec(C),
    )
    scratch = [pltpu.VMEM((B, NSLOT), jnp.float32)] * 3
    return pl.pallas_call(
        _big_kernel,
        grid=(NBLK,),
        in_specs=in_specs,
        out_specs=out_specs,
        out_shape=out_shapes,
        scratch_shapes=scratch,
    )(qf, q1, q2, mask2d, FM, M1, M2)


def _corr_kernel(feat_ref, s1_ref, s2_ref, qf_ref, q1_ref, q2_ref,
                 gfm_ref, gm1_ref, gm2_ref, vb_ref,
                 cf_ref, c1_ref, c2_ref, nf_ref, rdu_ref):
    nf = (1.0 - MOM) * gfm_ref[...] + MOM * feat_ref[...]
    n1 = (1.0 - MOM) * gm1_ref[...] + MOM * s1_ref[...]
    n2 = (1.0 - MOM) * gm2_ref[...] + MOM * s2_ref[...]
    nf_ref[...] = nf
    vbias = vb_ref[0:1, :]

    def corr(q, nr, out_ref):
        rn = jnp.sqrt(jnp.sum(nr * nr, axis=1))
        rinv = 1.0 / jnp.maximum(rn, EPS)
        cm = jax.lax.dot_general(q, nr, (((1,), (1,)), ((), ())),
                                 preferred_element_type=jnp.float32)
        out_ref[...] = cm * rinv[None, :] + vbias

    corr(qf_ref[...], nf, cf_ref)
    corr(q1_ref[...], n1, c1_ref)
    corr(q2_ref[...], n2, c2_ref)
    rdu = jnp.sum(jnp.abs(n1 - n2), axis=1)
    rdu_ref[...] = jnp.broadcast_to(rdu[None, :], (8, B))


@jax.jit
def _corr(feat, s1, s2, qf, q1, q2, gfm, gm1, gm2, vbias2d):
    out_shapes = (
        jax.ShapeDtypeStruct((B, B), jnp.float32),
        jax.ShapeDtypeStruct((B, B), jnp.float32),
        jax.ShapeDtypeStruct((B, B), jnp.float32),
        jax.ShapeDtypeStruct((B, D), jnp.float32),
        jax.ShapeDtypeStruct((8, B), jnp.float32),
    )
    return pl.pallas_call(_corr_kernel, out_shape=out_shapes)(
        feat, s1, s2, qf, q1, q2, gfm, gm1, gm2, vbias2d)


def _sel_kernel(cand_ref, corr_ref, pos_ref, w_ref):
    w_ref[:, :NCAND] = cand_ref[...]
    w_ref[:, NCAND:] = corr_ref[...]
    iota = jax.lax.broadcasted_iota(jnp.int32, (B, WTOT), 1)
    cols = []
    for _ in range(9):
        w = w_ref[...]
        m = jnp.max(w, axis=1)
        hit = w >= m[:, None]
        pos = jnp.min(jnp.where(hit, iota, BIGI), axis=1)
        cols.append(pos[:, None])
        w_ref[...] = jnp.where(hit, NEG, w)
    cols.append(jnp.zeros((B, NSLOT - 9), jnp.int32))
    pos_ref[...] = jnp.concatenate(cols, axis=1)


@jax.jit
def _sel(cand, corrm):
    return pl.pallas_call(
        _sel_kernel,
        out_shape=jax.ShapeDtypeStruct((B, NSLOT), jnp.int32),
        scratch_shapes=[pltpu.VMEM((B, WTOT), jnp.float32)],
    )(cand, corrm)


def _feat_loss_kernel(a_ref, b_ref, out_ref, acc_ref):
    i = pl.program_id(0)

    @pl.when(i == 0)
    def _():
        acc_ref[0] = 0.0

    a = a_ref[...]
    b = b_ref[...]
    ra = jnp.maximum(jnp.sqrt(jnp.sum(a * a, axis=1)), EPS)
    rb = jnp.maximum(jnp.sqrt(jnp.sum(b * b, axis=1)), EPS)
    dots = jnp.sum(a * b, axis=1) / (ra * rb)
    acc_ref[0] += jnp.sum(dots)

    @pl.when(i == pl.num_programs(0) - 1)
    def _():
        out_ref[0] = acc_ref[0]


@jax.jit
def _feat_loss(a, b):
    n = a.shape[0]
    blk = 1024
    grid = (n // blk,)
    return pl.pallas_call(
        _feat_loss_kernel,
        grid=grid,
        in_specs=[pl.BlockSpec((blk, D), lambda i: (i, 0))] * 2,
        out_specs=pl.BlockSpec(memory_space=pltpu.SMEM),
        out_shape=jax.ShapeDtypeStruct((1,), jnp.float32),
        scratch_shapes=[pltpu.SMEM((1,), jnp.float32)],
    )(a, b)


def kernel(target_feature, fc1, fc2, index_target, target_featurememory,
           target_softmaxFc1memory, target_softmaxFc2memory):
    feat = target_feature
    idx = index_target
    FM, M1, M2 = target_featurememory, target_softmaxFc1memory, target_softmaxFc2memory

    qf, s1, q1, s2, q2 = _prep(feat, fc1, fc2)

    jpos = jnp.arange(B, dtype=jnp.int32)
    w = jnp.full((T,), -1, dtype=jnp.int32).at[idx].max(jpos)
    valid = (w[idx] == jpos)
    vbias2d = jnp.broadcast_to(
        jnp.where(valid, 0.0, NEG)[None, :], (8, B))

    mask1 = jnp.zeros((TPAD,), jnp.float32).at[idx].set(NEG)
    mask2d = jnp.broadcast_to(mask1[None, :], (8, TPAD))

    Sf, S1, S2, bf, b1, b2, rd2d = _big(qf, q1, q2, mask2d, FM, M1, M2)
    rowdiff = rd2d[0, :T]

    gfm, gm1, gm2 = FM[idx], M1[idx], M2[idx]
    Cf, C1, C2, nf, rdu2d = _corr(feat, s1, s2, qf, q1, q2, gfm, gm1, gm2,
                                  vbias2d)
    rowdiff_upd = rdu2d[0, :]

    def topk9(S, blks, Cm):
        blk9 = blks[:, :9]
        Sb = S.reshape(B, NBLK, TBLK)
        cand = jnp.take_along_axis(Sb, blk9[:, :, None], axis=1).reshape(B, NCAND)
        pos = _sel(cand, Cm)[:, :9]
        incand = pos < NCAND
        pc = jnp.minimum(pos, NCAND - 1)
        gblk = jnp.take_along_axis(blk9, pc // TBLK, axis=1)
        gc = gblk * TBLK + pc % TBLK
        gr = idx[jnp.clip(pos - NCAND, 0, B - 1)]
        return jnp.where(incand, gc, gr)

    f_idx = topk9(Sf, bf, Cf)[:, 1:9].reshape(-1)
    i1 = topk9(S1, b1, C1)[:, 1:8].reshape(-1)
    i2 = topk9(S2, b2, C2)[:, 1:8].reshape(-1)

    posf = w[f_idx]
    vals = jnp.where(posf >= 0, rowdiff_upd[jnp.maximum(posf, 0)], rowdiff[f_idx])
    loss_softmax = jnp.sum(vals) / (B * 8 * C)

    def fetch(ii):
        p = w[ii]
        base = FM[ii]
        upd = nf[jnp.maximum(p, 0)]
        return jnp.where((p >= 0)[:, None], upd, base)

    a = fetch(i1)
    b = fetch(i2)
    n = i1.shape[0]
    dotsum = _feat_loss(a, b)[0]
    loss_feature = 0.5 - dotsum / (2.0 * n)

    return (loss_softmax, loss_feature)


# speedup vs baseline: 37.0473x; 1.2422x over previous
import functools

import jax
import jax.numpy as jnp
from jax import lax
from jax.experimental import pallas as pl
from jax.experimental.pallas import tpu as pltpu
from jax.experimental.pallas import tpu_sc as plsc

B, D, C, T = 1024, 512, 1000, 50000
TOP = 8
MOM = 0.1
TBLK = 512
NBLK = 98
TPAD = NBLK * TBLK
EPS = 1e-12
NEG = -jnp.inf
NSLOT = 16
NCAND = 9 * TBLK
WTOT = NCAND + B
BIGI = 2 ** 30


def _norm_rows(x):
    n = jnp.sqrt(jnp.sum(x * x, axis=1, keepdims=True))
    return x / jnp.maximum(n, EPS)


def _prep_kernel(feat_ref, fc1_ref, fc2_ref, qf_ref, s1_ref, q1_ref,
                 s2_ref, q2_ref):
    feat = feat_ref[...]
    qf_ref[...] = _norm_rows(feat)

    def soft(x):
        m = jnp.max(x, axis=1, keepdims=True)
        e = jnp.exp(x - m)
        return e / jnp.sum(e, axis=1, keepdims=True)

    s1 = soft(fc1_ref[...])
    s1_ref[...] = s1
    q1_ref[...] = _norm_rows(s1)
    s2 = soft(fc2_ref[...])
    s2_ref[...] = s2
    q2_ref[...] = _norm_rows(s2)


@jax.jit
def _prep(feat, fc1, fc2):
    shp = lambda c: jax.ShapeDtypeStruct((B, c), jnp.float32)
    return pl.pallas_call(
        _prep_kernel,
        out_shape=(shp(D), shp(C), shp(C), shp(C), shp(C)),
    )(feat, fc1, fc2)


def _sim_block(q, mem, maskrow, colvalid):
    rsq = jnp.sum(mem * mem, axis=1)
    rinv = 1.0 / jnp.maximum(jnp.sqrt(rsq), EPS)
    s = jax.lax.dot_general(q, mem, (((1,), (1,)), ((), ())),
                            preferred_element_type=jnp.float32)
    s = s * rinv[None, :] + maskrow
    return jnp.where(colvalid, s, NEG)


def _merge_top9(step, bm, vals_ref, blks_ref):
    slots = jax.lax.broadcasted_iota(jnp.int32, (B, NSLOT), 1)

    @pl.when(step == 0)
    def _():
        vals_ref[...] = jnp.where(slots < 9, NEG, jnp.inf)
        blks_ref[...] = jnp.zeros((B, NSLOT), jnp.int32)

    vals = vals_ref[...]
    cmin = jnp.min(vals, axis=1)
    sel = jnp.where(vals == cmin[:, None], slots, NSLOT + 1)
    p = jnp.min(sel, axis=1)
    hit = (slots == p[:, None]) & (bm > cmin)[:, None]
    vals_ref[...] = jnp.where(hit, bm[:, None], vals)
    blks_ref[...] = jnp.where(hit, step, blks_ref[...])


def _big_kernel(qf_ref, q1_ref, q2_ref, mask_ref, fm_ref, m1_ref, m2_ref,
                sf_ref, s1_ref, s2_ref, bf_ref, b1_ref, b2_ref, rd_ref,
                vf_ref, v1_ref, v2_ref):
    i = pl.program_id(0)
    maskrow = mask_ref[0:1, :]
    col = jax.lax.broadcasted_iota(jnp.int32, (B, TBLK), 1) + i * TBLK
    colvalid = col < T

    sf = _sim_block(qf_ref[...], fm_ref[...], maskrow, colvalid)
    sf_ref[...] = sf
    _merge_top9(i, jnp.max(sf, axis=1), vf_ref, bf_ref)

    m1 = m1_ref[...]
    m2 = m2_ref[...]
    s1 = _sim_block(q1_ref[...], m1, maskrow, colvalid)
    s1_ref[...] = s1
    _merge_top9(i, jnp.max(s1, axis=1), v1_ref, b1_ref)

    s2 = _sim_block(q2_ref[...], m2, maskrow, colvalid)
    s2_ref[...] = s2
    _merge_top9(i, jnp.max(s2, axis=1), v2_ref, b2_ref)

    rd = jnp.sum(jnp.abs(m1 - m2), axis=1)
    rd_ref[...] = jnp.broadcast_to(rd[None, :], (8, TBLK))


@jax.jit
def _big(qf, q1, q2, mask2d, FM, M1, M2):
    res_spec = lambda shape: pl.BlockSpec(shape, lambda i: (0, 0))
    mem_spec = lambda width: pl.BlockSpec((TBLK, width), lambda i: (i, 0))
    s_spec = pl.BlockSpec((B, TBLK), lambda i: (0, i))
    out_shapes = (
        jax.ShapeDtypeStruct((B, TPAD), jnp.float32),
        jax.ShapeDtypeStruct((B, TPAD), jnp.float32),
        jax.ShapeDtypeStruct((B, TPAD), jnp.float32),
        jax.ShapeDtypeStruct((B, NSLOT), jnp.int32),
        jax.ShapeDtypeStruct((B, NSLOT), jnp.int32),
        jax.ShapeDtypeStruct((B, NSLOT), jnp.int32),
        jax.ShapeDtypeStruct((8, TPAD), jnp.float32),
    )
    out_specs = (
        s_spec, s_spec, s_spec,
        res_spec((B, NSLOT)), res_spec((B, NSLOT)), res_spec((B, NSLOT)),
        pl.BlockSpec((8, TBLK), lambda i: (0, i)),
    )
    in_specs = (
        res_spec((B, D)),
        res_spec((B, C)),
        res_spec((B, C)),
        pl.BlockSpec((8, TBLK), lambda i: (0, i)),
        mem_spec(D), mem_spec(C), mem_spec(C),
    )
    scratch = [pltpu.VMEM((B, NSLOT), jnp.float32)] * 3
    return pl.pallas_call(
        _big_kernel,
        grid=(NBLK,),
        in_specs=in_specs,
        out_specs=out_specs,
        out_shape=out_shapes,
        scratch_shapes=scratch,
    )(qf, q1, q2, mask2d, FM, M1, M2)


def _corr_kernel(feat_ref, s1_ref, s2_ref, qf_ref, q1_ref, q2_ref,
                 gfm_ref, gm1_ref, gm2_ref, vb_ref,
                 cf_ref, c1_ref, c2_ref, nf_ref, rdu_ref):
    nf = (1.0 - MOM) * gfm_ref[...] + MOM * feat_ref[...]
    n1 = (1.0 - MOM) * gm1_ref[...] + MOM * s1_ref[...]
    n2 = (1.0 - MOM) * gm2_ref[...] + MOM * s2_ref[...]
    nf_ref[...] = nf
    vbias = vb_ref[0:1, :]

    def corr(q, nr, out_ref):
        rn = jnp.sqrt(jnp.sum(nr * nr, axis=1))
        rinv = 1.0 / jnp.maximum(rn, EPS)
        cm = jax.lax.dot_general(q, nr, (((1,), (1,)), ((), ())),
                                 preferred_element_type=jnp.float32)
        out_ref[...] = cm * rinv[None, :] + vbias

    corr(qf_ref[...], nf, cf_ref)
    corr(q1_ref[...], n1, c1_ref)
    corr(q2_ref[...], n2, c2_ref)
    rdu = jnp.sum(jnp.abs(n1 - n2), axis=1)
    rdu_ref[...] = jnp.broadcast_to(rdu[None, :], (8, B))


@jax.jit
def _corr(feat, s1, s2, qf, q1, q2, gfm, gm1, gm2, vbias2d):
    out_shapes = (
        jax.ShapeDtypeStruct((B, B), jnp.float32),
        jax.ShapeDtypeStruct((B, B), jnp.float32),
        jax.ShapeDtypeStruct((B, B), jnp.float32),
        jax.ShapeDtypeStruct((B, D), jnp.float32),
        jax.ShapeDtypeStruct((8, B), jnp.float32),
    )
    return pl.pallas_call(_corr_kernel, out_shape=out_shapes)(
        feat, s1, s2, qf, q1, q2, gfm, gm1, gm2, vbias2d)


def _sel_kernel(cand_ref, corr_ref, pos_ref, w_ref):
    w_ref[:, :NCAND] = cand_ref[...]
    w_ref[:, NCAND:] = corr_ref[...]
    iota = jax.lax.broadcasted_iota(jnp.int32, (B, WTOT), 1)
    cols = []
    for _ in range(9):
        w = w_ref[...]
        m = jnp.max(w, axis=1)
        hit = w >= m[:, None]
        pos = jnp.min(jnp.where(hit, iota, BIGI), axis=1)
        cols.append(pos[:, None])
        w_ref[...] = jnp.where(hit, NEG, w)
    cols.append(jnp.zeros((B, NSLOT - 9), jnp.int32))
    pos_ref[...] = jnp.concatenate(cols, axis=1)


@jax.jit
def _sel(cand, corrm):
    return pl.pallas_call(
        _sel_kernel,
        out_shape=jax.ShapeDtypeStruct((B, NSLOT), jnp.int32),
        scratch_shapes=[pltpu.VMEM((B, WTOT), jnp.float32)],
    )(cand, corrm)


NW = 32
NROWS = 9 * B
ROWS_PER_W = NROWS // NW
SC_CHUNK = 96


@functools.partial(
    pl.kernel,
    mesh=plsc.VectorSubcoreMesh(core_axis_name="c", subcore_axis_name="s"),
    out_type=[jax.ShapeDtypeStruct((NROWS, TBLK), jnp.float32)] * 3,
    scratch_types=[
        pltpu.VMEM((SC_CHUNK,), jnp.int32),
        pltpu.VMEM((SC_CHUNK, TBLK), jnp.float32),
        pltpu.SemaphoreType.DMA,
    ],
)
def _sc_cand_gather(sf, s1, s2, jf, j1, j2, of, o1, o2, idx_v, rows_v, sem):
    wid = lax.axis_index("s") * 2 + lax.axis_index("c")
    for tab, ind, out in ((sf, jf, of), (s1, j1, o1), (s2, j2, o2)):
        for c in range(ROWS_PER_W // SC_CHUNK):
            base = wid * ROWS_PER_W + c * SC_CHUNK
            pltpu.sync_copy(ind.at[pl.ds(base, SC_CHUNK)], idx_v)
            pltpu.async_copy(tab.at[idx_v], rows_v, sem).wait()
            pltpu.sync_copy(rows_v, out.at[pl.ds(base, SC_CHUNK)])


def _feat_loss_kernel(a_ref, b_ref, out_ref, acc_ref):
    i = pl.program_id(0)

    @pl.when(i == 0)
    def _():
        acc_ref[0] = 0.0

    a = a_ref[...]
    b = b_ref[...]
    ra = jnp.maximum(jnp.sqrt(jnp.sum(a * a, axis=1)), EPS)
    rb = jnp.maximum(jnp.sqrt(jnp.sum(b * b, axis=1)), EPS)
    dots = jnp.sum(a * b, axis=1) / (ra * rb)
    acc_ref[0] += jnp.sum(dots)

    @pl.when(i == pl.num_programs(0) - 1)
    def _():
        out_ref[0] = acc_ref[0]


@jax.jit
def _feat_loss(a, b):
    n = a.shape[0]
    blk = 1024
    grid = (n // blk,)
    return pl.pallas_call(
        _feat_loss_kernel,
        grid=grid,
        in_specs=[pl.BlockSpec((blk, D), lambda i: (i, 0))] * 2,
        out_specs=pl.BlockSpec(memory_space=pltpu.SMEM),
        out_shape=jax.ShapeDtypeStruct((1,), jnp.float32),
        scratch_shapes=[pltpu.SMEM((1,), jnp.float32)],
    )(a, b)


def kernel(target_feature, fc1, fc2, index_target, target_featurememory,
           target_softmaxFc1memory, target_softmaxFc2memory):
    feat = target_feature
    idx = index_target
    FM, M1, M2 = target_featurememory, target_softmaxFc1memory, target_softmaxFc2memory

    qf, s1, q1, s2, q2 = _prep(feat, fc1, fc2)

    jpos = jnp.arange(B, dtype=jnp.int32)
    w = jnp.full((T,), -1, dtype=jnp.int32).at[idx].max(jpos)
    valid = (w[idx] == jpos)
    vbias2d = jnp.broadcast_to(
        jnp.where(valid, 0.0, NEG)[None, :], (8, B))

    mask1 = jnp.zeros((TPAD,), jnp.float32).at[idx].set(NEG)
    mask2d = jnp.broadcast_to(mask1[None, :], (8, TPAD))

    Sf, S1, S2, bf, b1, b2, rd2d = _big(qf, q1, q2, mask2d, FM, M1, M2)
    rowdiff = rd2d[0, :T]

    gfm, gm1, gm2 = FM[idx], M1[idx], M2[idx]
    Cf, C1, C2, nf, rdu2d = _corr(feat, s1, s2, qf, q1, q2, gfm, gm1, gm2,
                                  vbias2d)
    rowdiff_upd = rdu2d[0, :]

    blk9_f, blk9_1, blk9_2 = bf[:, :9], b1[:, :9], b2[:, :9]
    rowbase = jnp.arange(B, dtype=jnp.int32)[:, None] * NBLK
    jf = (rowbase + blk9_f).reshape(-1)
    j1 = (rowbase + blk9_1).reshape(-1)
    j2 = (rowbase + blk9_2).reshape(-1)
    cand_f, cand_1, cand_2 = _sc_cand_gather(
        Sf.reshape(B * NBLK, TBLK), S1.reshape(B * NBLK, TBLK),
        S2.reshape(B * NBLK, TBLK), jf, j1, j2)

    def topk9(cand, blk9, Cm):
        cand = cand.reshape(B, NCAND)
        pos = _sel(cand, Cm)[:, :9]
        incand = pos < NCAND
        pc = jnp.minimum(pos, NCAND - 1)
        gblk = jnp.take_along_axis(blk9, pc // TBLK, axis=1)
        gc = gblk * TBLK + pc % TBLK
        gr = idx[jnp.clip(pos - NCAND, 0, B - 1)]
        return jnp.where(incand, gc, gr)

    f_idx = topk9(cand_f, blk9_f, Cf)[:, 1:9].reshape(-1)
    i1 = topk9(cand_1, blk9_1, C1)[:, 1:8].reshape(-1)
    i2 = topk9(cand_2, blk9_2, C2)[:, 1:8].reshape(-1)

    posf = w[f_idx]
    vals = jnp.where(posf >= 0, rowdiff_upd[jnp.maximum(posf, 0)], rowdiff[f_idx])
    loss_softmax = jnp.sum(vals) / (B * 8 * C)

    def fetch(ii):
        p = w[ii]
        base = FM[ii]
        upd = nf[jnp.maximum(p, 0)]
        return jnp.where((p >= 0)[:, None], upd, base)

    a = fetch(i1)
    b = fetch(i2)
    n = i1.shape[0]
    dotsum = _feat_loss(a, b)[0]
    loss_feature = 0.5 - dotsum / (2.0 * n)

    return (loss_softmax, loss_feature)


# speedup vs baseline: 50.2261x; 1.3557x over previous
import functools

import jax
import jax.numpy as jnp
from jax import lax
from jax.experimental import pallas as pl
from jax.experimental.pallas import tpu as pltpu
from jax.experimental.pallas import tpu_sc as plsc

B, D, C, T = 1024, 512, 1000, 50000
TOP = 8
MOM = 0.1
TBLK = 512
NBLK = 98
TPAD = NBLK * TBLK
EPS = 1e-12
NEG = -jnp.inf
NSLOT = 16
NCAND = 9 * TBLK
WTOT = NCAND + B
BIGI = 2 ** 30


def _norm_rows(x):
    n = jnp.sqrt(jnp.sum(x * x, axis=1, keepdims=True))
    return x / jnp.maximum(n, EPS)


def _prep_kernel(feat_ref, fc1_ref, fc2_ref, qf_ref, s1_ref, q1_ref,
                 s2_ref, q2_ref):
    feat = feat_ref[...]
    qf_ref[...] = _norm_rows(feat)

    def soft(x):
        m = jnp.max(x, axis=1, keepdims=True)
        e = jnp.exp(x - m)
        return e / jnp.sum(e, axis=1, keepdims=True)

    s1 = soft(fc1_ref[...])
    s1_ref[...] = s1
    q1_ref[...] = _norm_rows(s1)
    s2 = soft(fc2_ref[...])
    s2_ref[...] = s2
    q2_ref[...] = _norm_rows(s2)


@jax.jit
def _prep(feat, fc1, fc2):
    shp = lambda c: jax.ShapeDtypeStruct((B, c), jnp.float32)
    return pl.pallas_call(
        _prep_kernel,
        out_shape=(shp(D), shp(C), shp(C), shp(C), shp(C)),
    )(feat, fc1, fc2)


def _sim_block(q, mem, maskrow, colvalid):
    rsq = jnp.sum(mem * mem, axis=1)
    rinv = 1.0 / jnp.maximum(jnp.sqrt(rsq), EPS)
    s = jax.lax.dot_general(q, mem, (((1,), (1,)), ((), ())),
                            preferred_element_type=jnp.float32)
    s = s * rinv[None, :] + maskrow
    return jnp.where(colvalid, s, NEG)


def _merge_top9(step, bm, vals_ref, blks_ref):
    slots = jax.lax.broadcasted_iota(jnp.int32, (B, NSLOT), 1)

    @pl.when(step == 0)
    def _():
        vals_ref[...] = jnp.where(slots < 9, NEG, jnp.inf)
        blks_ref[...] = jnp.zeros((B, NSLOT), jnp.int32)

    vals = vals_ref[...]
    cmin = jnp.min(vals, axis=1)
    sel = jnp.where(vals == cmin[:, None], slots, NSLOT + 1)
    p = jnp.min(sel, axis=1)
    hit = (slots == p[:, None]) & (bm > cmin)[:, None]
    vals_ref[...] = jnp.where(hit, bm[:, None], vals)
    blks_ref[...] = jnp.where(hit, step, blks_ref[...])


def _big_kernel(qf_ref, q1_ref, q2_ref, mask_ref, fm_ref, m1_ref, m2_ref,
                sf_ref, s1_ref, s2_ref, bf_ref, b1_ref, b2_ref, rd_ref,
                vf_ref, v1_ref, v2_ref):
    i = pl.program_id(0)
    maskrow = mask_ref[0:1, :]
    col = jax.lax.broadcasted_iota(jnp.int32, (B, TBLK), 1) + i * TBLK
    colvalid = col < T

    sf = _sim_block(qf_ref[...], fm_ref[...], maskrow, colvalid)
    sf_ref[0] = sf
    _merge_top9(i, jnp.max(sf, axis=1), vf_ref, bf_ref)

    m1 = m1_ref[...]
    m2 = m2_ref[...]
    s1 = _sim_block(q1_ref[...], m1, maskrow, colvalid)
    s1_ref[0] = s1
    _merge_top9(i, jnp.max(s1, axis=1), v1_ref, b1_ref)

    s2 = _sim_block(q2_ref[...], m2, maskrow, colvalid)
    s2_ref[0] = s2
    _merge_top9(i, jnp.max(s2, axis=1), v2_ref, b2_ref)

    rd = jnp.sum(jnp.abs(m1 - m2), axis=1)
    rd_ref[...] = jnp.broadcast_to(rd[None, :], (8, TBLK))


@jax.jit
def _big(qf, q1, q2, mask2d, FM, M1, M2):
    res_spec = lambda shape: pl.BlockSpec(shape, lambda i: (0, 0))
    mem_spec = lambda width: pl.BlockSpec((TBLK, width), lambda i: (i, 0))
    s_spec = pl.BlockSpec((1, B, TBLK), lambda i: (i, 0, 0))
    out_shapes = (
        jax.ShapeDtypeStruct((NBLK, B, TBLK), jnp.float32),
        jax.ShapeDtypeStruct((NBLK, B, TBLK), jnp.float32),
        jax.ShapeDtypeStruct((NBLK, B, TBLK), jnp.float32),
        jax.ShapeDtypeStruct((B, NSLOT), jnp.int32),
        jax.ShapeDtypeStruct((B, NSLOT), jnp.int32),
        jax.ShapeDtypeStruct((B, NSLOT), jnp.int32),
        jax.ShapeDtypeStruct((8, TPAD), jnp.float32),
    )
    out_specs = (
        s_spec, s_spec, s_spec,
        res_spec((B, NSLOT)), res_spec((B, NSLOT)), res_spec((B, NSLOT)),
        pl.BlockSpec((8, TBLK), lambda i: (0, i)),
    )
    in_specs = (
        res_spec((B, D)),
        res_spec((B, C)),
        res_spec((B, C)),
        pl.BlockSpec((8, TBLK), lambda i: (0, i)),
        mem_spec(D), mem_spec(C), mem_spec(C),
    )
    scratch = [pltpu.VMEM((B, NSLOT), jnp.float32)] * 3
    return pl.pallas_call(
        _big_kernel,
        grid=(NBLK,),
        in_specs=in_specs,
        out_specs=out_specs,
        out_shape=out_shapes,
        scratch_shapes=scratch,
    )(qf, q1, q2, mask2d, FM, M1, M2)


def _corr_kernel(feat_ref, s1_ref, s2_ref, qf_ref, q1_ref, q2_ref,
                 gfm_ref, gm1_ref, gm2_ref, vb_ref,
                 cf_ref, c1_ref, c2_ref, nf_ref, rdu_ref):
    nf = (1.0 - MOM) * gfm_ref[...] + MOM * feat_ref[...]
    n1 = (1.0 - MOM) * gm1_ref[...] + MOM * s1_ref[...]
    n2 = (1.0 - MOM) * gm2_ref[...] + MOM * s2_ref[...]
    nf_ref[...] = nf
    vbias = vb_ref[0:1, :]

    def corr(q, nr, out_ref):
        rn = jnp.sqrt(jnp.sum(nr * nr, axis=1))
        rinv = 1.0 / jnp.maximum(rn, EPS)
        cm = jax.lax.dot_general(q, nr, (((1,), (1,)), ((), ())),
                                 preferred_element_type=jnp.float32)
        out_ref[...] = cm * rinv[None, :] + vbias

    corr(qf_ref[...], nf, cf_ref)
    corr(q1_ref[...], n1, c1_ref)
    corr(q2_ref[...], n2, c2_ref)
    rdu = jnp.sum(jnp.abs(n1 - n2), axis=1)
    rdu_ref[...] = jnp.broadcast_to(rdu[None, :], (8, B))


@jax.jit
def _corr(feat, s1, s2, qf, q1, q2, gfm, gm1, gm2, vbias2d):
    out_shapes = (
        jax.ShapeDtypeStruct((B, B), jnp.float32),
        jax.ShapeDtypeStruct((B, B), jnp.float32),
        jax.ShapeDtypeStruct((B, B), jnp.float32),
        jax.ShapeDtypeStruct((B, D), jnp.float32),
        jax.ShapeDtypeStruct((8, B), jnp.float32),
    )
    return pl.pallas_call(_corr_kernel, out_shape=out_shapes)(
        feat, s1, s2, qf, q1, q2, gfm, gm1, gm2, vbias2d)


def _sel_kernel(cand_ref, corr_ref, pos_ref, w_ref):
    for k in range(9):
        w_ref[:, k * TBLK:(k + 1) * TBLK] = cand_ref[k]
    w_ref[:, NCAND:] = corr_ref[...]
    iota = jax.lax.broadcasted_iota(jnp.int32, (B, WTOT), 1)
    cols = []
    for _ in range(9):
        w = w_ref[...]
        m = jnp.max(w, axis=1)
        hit = w >= m[:, None]
        pos = jnp.min(jnp.where(hit, iota, BIGI), axis=1)
        cols.append(pos[:, None])
        w_ref[...] = jnp.where(hit, NEG, w)
    cols.append(jnp.zeros((B, NSLOT - 9), jnp.int32))
    pos_ref[...] = jnp.concatenate(cols, axis=1)


@jax.jit
def _sel(cand3d, corrm):
    return pl.pallas_call(
        _sel_kernel,
        out_shape=jax.ShapeDtypeStruct((B, NSLOT), jnp.int32),
        scratch_shapes=[pltpu.VMEM((B, WTOT), jnp.float32)],
    )(cand3d, corrm)


NW = 32
NROWS = 9 * B
ROWS_PER_W = NROWS // NW
SC_CHUNK = 96


@functools.partial(
    pl.kernel,
    mesh=plsc.VectorSubcoreMesh(core_axis_name="c", subcore_axis_name="s"),
    out_type=[jax.ShapeDtypeStruct((NROWS, TBLK), jnp.float32)] * 3,
    scratch_types=[
        pltpu.VMEM((SC_CHUNK,), jnp.int32),
        pltpu.VMEM((SC_CHUNK, TBLK), jnp.float32),
        pltpu.SemaphoreType.DMA,
    ],
)
def _sc_cand_gather(sf, s1, s2, jf, j1, j2, of, o1, o2, idx_v, rows_v, sem):
    wid = lax.axis_index("s") * 2 + lax.axis_index("c")
    for tab, ind, out in ((sf, jf, of), (s1, j1, o1), (s2, j2, o2)):
        for c in range(ROWS_PER_W // SC_CHUNK):
            base = wid * ROWS_PER_W + c * SC_CHUNK
            pltpu.sync_copy(ind.at[pl.ds(base, SC_CHUNK)], idx_v)
            pltpu.async_copy(tab.at[idx_v], rows_v, sem).wait()
            pltpu.sync_copy(rows_v, out.at[pl.ds(base, SC_CHUNK)])


def _feat_loss_kernel(a_ref, b_ref, out_ref, acc_ref):
    i = pl.program_id(0)

    @pl.when(i == 0)
    def _():
        acc_ref[0] = 0.0

    a = a_ref[...]
    b = b_ref[...]
    ra = jnp.maximum(jnp.sqrt(jnp.sum(a * a, axis=1)), EPS)
    rb = jnp.maximum(jnp.sqrt(jnp.sum(b * b, axis=1)), EPS)
    dots = jnp.sum(a * b, axis=1) / (ra * rb)
    acc_ref[0] += jnp.sum(dots)

    @pl.when(i == pl.num_programs(0) - 1)
    def _():
        out_ref[0] = acc_ref[0]


@jax.jit
def _feat_loss(a, b):
    n = a.shape[0]
    blk = 1024
    grid = (n // blk,)
    return pl.pallas_call(
        _feat_loss_kernel,
        grid=grid,
        in_specs=[pl.BlockSpec((blk, D), lambda i: (i, 0))] * 2,
        out_specs=pl.BlockSpec(memory_space=pltpu.SMEM),
        out_shape=jax.ShapeDtypeStruct((1,), jnp.float32),
        scratch_shapes=[pltpu.SMEM((1,), jnp.float32)],
    )(a, b)


def kernel(target_feature, fc1, fc2, index_target, target_featurememory,
           target_softmaxFc1memory, target_softmaxFc2memory):
    feat = target_feature
    idx = index_target
    FM, M1, M2 = target_featurememory, target_softmaxFc1memory, target_softmaxFc2memory

    qf, s1, q1, s2, q2 = _prep(feat, fc1, fc2)

    jpos = jnp.arange(B, dtype=jnp.int32)
    w = jnp.full((T,), -1, dtype=jnp.int32).at[idx].max(jpos)
    valid = (w[idx] == jpos)
    vbias2d = jnp.broadcast_to(
        jnp.where(valid, 0.0, NEG)[None, :], (8, B))

    mask1 = jnp.zeros((TPAD,), jnp.float32).at[idx].set(NEG)
    mask2d = jnp.broadcast_to(mask1[None, :], (8, TPAD))

    Sf, S1, S2, bf, b1, b2, rd2d = _big(qf, q1, q2, mask2d, FM, M1, M2)
    rowdiff = rd2d[0, :T]

    gfm, gm1, gm2 = FM[idx], M1[idx], M2[idx]
    Cf, C1, C2, nf, rdu2d = _corr(feat, s1, s2, qf, q1, q2, gfm, gm1, gm2,
                                  vbias2d)
    rowdiff_upd = rdu2d[0, :]

    blk9_f, blk9_1, blk9_2 = bf[:, :9], b1[:, :9], b2[:, :9]
    rowid = jnp.arange(B, dtype=jnp.int32)[:, None]
    jf = (blk9_f * B + rowid).T.reshape(-1)
    j1 = (blk9_1 * B + rowid).T.reshape(-1)
    j2 = (blk9_2 * B + rowid).T.reshape(-1)
    cand_f, cand_1, cand_2 = _sc_cand_gather(
        Sf.reshape(NBLK * B, TBLK), S1.reshape(NBLK * B, TBLK),
        S2.reshape(NBLK * B, TBLK), jf, j1, j2)

    def topk9(cand, blk9, Cm):
        pos = _sel(cand.reshape(9, B, TBLK), Cm)[:, :9]
        incand = pos < NCAND
        pc = jnp.minimum(pos, NCAND - 1)
        gblk = jnp.take_along_axis(blk9, pc // TBLK, axis=1)
        gc = gblk * TBLK + pc % TBLK
        gr = idx[jnp.clip(pos - NCAND, 0, B - 1)]
        return jnp.where(incand, gc, gr)

    f_idx = topk9(cand_f, blk9_f, Cf)[:, 1:9].reshape(-1)
    i1 = topk9(cand_1, blk9_1, C1)[:, 1:8].reshape(-1)
    i2 = topk9(cand_2, blk9_2, C2)[:, 1:8].reshape(-1)

    posf = w[f_idx]
    vals = jnp.where(posf >= 0, rowdiff_upd[jnp.maximum(posf, 0)], rowdiff[f_idx])
    loss_softmax = jnp.sum(vals) / (B * 8 * C)

    def fetch(ii):
        p = w[ii]
        base = FM[ii]
        upd = nf[jnp.maximum(p, 0)]
        return jnp.where((p >= 0)[:, None], upd, base)

    a = fetch(i1)
    b = fetch(i2)
    n = i1.shape[0]
    dotsum = _feat_loss(a, b)[0]
    loss_feature = 0.5 - dotsum / (2.0 * n)

    return (loss_softmax, loss_feature)


# speedup vs baseline: 50.3452x; 1.0024x over previous
import functools

import jax
import jax.numpy as jnp
from jax import lax
from jax.experimental import pallas as pl
from jax.experimental.pallas import tpu as pltpu
from jax.experimental.pallas import tpu_sc as plsc

B, D, C, T = 1024, 512, 1000, 50000
TOP = 8
MOM = 0.1
TBLK = 512
NBLK = 98
TPAD = NBLK * TBLK
EPS = 1e-12
NEG = -jnp.inf
NSLOT = 16
NCAND = 9 * TBLK
WTOT = NCAND + B
BIGI = 2 ** 30


def _norm_rows(x):
    n = jnp.sqrt(jnp.sum(x * x, axis=1, keepdims=True))
    return x / jnp.maximum(n, EPS)


def _prep_kernel(feat_ref, fc1_ref, fc2_ref, qf_ref, s1_ref, q1_ref,
                 s2_ref, q2_ref):
    feat = feat_ref[...]
    qf_ref[...] = _norm_rows(feat)

    def soft(x):
        m = jnp.max(x, axis=1, keepdims=True)
        e = jnp.exp(x - m)
        return e / jnp.sum(e, axis=1, keepdims=True)

    s1 = soft(fc1_ref[...])
    s1_ref[...] = s1
    q1_ref[...] = _norm_rows(s1)
    s2 = soft(fc2_ref[...])
    s2_ref[...] = s2
    q2_ref[...] = _norm_rows(s2)


@jax.jit
def _prep(feat, fc1, fc2):
    shp = lambda c: jax.ShapeDtypeStruct((B, c), jnp.float32)
    return pl.pallas_call(
        _prep_kernel,
        out_shape=(shp(D), shp(C), shp(C), shp(C), shp(C)),
    )(feat, fc1, fc2)


def _sim_block(q, mem, maskrow, colvalid, mm_dtype=jnp.float32):
    rsq = jnp.sum(mem * mem, axis=1)
    rinv = 1.0 / jnp.maximum(jnp.sqrt(rsq), EPS)
    s = jax.lax.dot_general(q.astype(mm_dtype), mem.astype(mm_dtype),
                            (((1,), (1,)), ((), ())),
                            preferred_element_type=jnp.float32)
    s = s * rinv[None, :] + maskrow
    return jnp.where(colvalid, s, NEG)


def _merge_top9(step, bm, vals_ref, blks_ref):
    slots = jax.lax.broadcasted_iota(jnp.int32, (B, NSLOT), 1)

    @pl.when(step == 0)
    def _():
        vals_ref[...] = jnp.where(slots < 9, NEG, jnp.inf)
        blks_ref[...] = jnp.zeros((B, NSLOT), jnp.int32)

    vals = vals_ref[...]
    cmin = jnp.min(vals, axis=1)
    sel = jnp.where(vals == cmin[:, None], slots, NSLOT + 1)
    p = jnp.min(sel, axis=1)
    hit = (slots == p[:, None]) & (bm > cmin)[:, None]
    vals_ref[...] = jnp.where(hit, bm[:, None], vals)
    blks_ref[...] = jnp.where(hit, step, blks_ref[...])


def _make_big(s_dtype, mm_dtype):
    def body(qf_ref, q1_ref, q2_ref, mask_ref, fm_ref, m1_ref, m2_ref,
             sf_ref, s1_ref, s2_ref, bf_ref, b1_ref, b2_ref, rd_ref,
             vf_ref, v1_ref, v2_ref):
        i = pl.program_id(0)
        maskrow = mask_ref[0:1, :]
        col = jax.lax.broadcasted_iota(jnp.int32, (B, TBLK), 1) + i * TBLK
        colvalid = col < T

        sf = _sim_block(qf_ref[...], fm_ref[...], maskrow, colvalid, mm_dtype)
        sf_ref[0] = sf.astype(s_dtype)
        _merge_top9(i, jnp.max(sf, axis=1), vf_ref, bf_ref)

        m1 = m1_ref[...]
        m2 = m2_ref[...]
        s1 = _sim_block(q1_ref[...], m1, maskrow, colvalid, mm_dtype)
        s1_ref[0] = s1.astype(s_dtype)
        _merge_top9(i, jnp.max(s1, axis=1), v1_ref, b1_ref)

        s2 = _sim_block(q2_ref[...], m2, maskrow, colvalid, mm_dtype)
        s2_ref[0] = s2.astype(s_dtype)
        _merge_top9(i, jnp.max(s2, axis=1), v2_ref, b2_ref)

        rd = jnp.sum(jnp.abs(m1 - m2), axis=1)
        rd_ref[...] = jnp.broadcast_to(rd[None, :], (8, TBLK))

    res_spec = lambda shape: pl.BlockSpec(shape, lambda i: (0, 0))
    mem_spec = lambda width: pl.BlockSpec((TBLK, width), lambda i: (i, 0))
    s_spec = pl.BlockSpec((1, B, TBLK), lambda i: (i, 0, 0))
    out_shapes = (
        jax.ShapeDtypeStruct((NBLK, B, TBLK), s_dtype),
        jax.ShapeDtypeStruct((NBLK, B, TBLK), s_dtype),
        jax.ShapeDtypeStruct((NBLK, B, TBLK), s_dtype),
        jax.ShapeDtypeStruct((B, NSLOT), jnp.int32),
        jax.ShapeDtypeStruct((B, NSLOT), jnp.int32),
        jax.ShapeDtypeStruct((B, NSLOT), jnp.int32),
        jax.ShapeDtypeStruct((8, TPAD), jnp.float32),
    )
    out_specs = (
        s_spec, s_spec, s_spec,
        res_spec((B, NSLOT)), res_spec((B, NSLOT)), res_spec((B, NSLOT)),
        pl.BlockSpec((8, TBLK), lambda i: (0, i)),
    )
    in_specs = (
        res_spec((B, D)),
        res_spec((B, C)),
        res_spec((B, C)),
        pl.BlockSpec((8, TBLK), lambda i: (0, i)),
        mem_spec(D), mem_spec(C), mem_spec(C),
    )
    scratch = [pltpu.VMEM((B, NSLOT), jnp.float32)] * 3
    return jax.jit(pl.pallas_call(
        body,
        grid=(NBLK,),
        in_specs=in_specs,
        out_specs=out_specs,
        out_shape=out_shapes,
        scratch_shapes=scratch,
    ))


_big = _make_big(jnp.float32, jnp.float32)


def _corr_kernel(feat_ref, s1_ref, s2_ref, qf_ref, q1_ref, q2_ref,
                 gfm_ref, gm1_ref, gm2_ref, vb_ref,
                 cf_ref, c1_ref, c2_ref, nf_ref, rdu_ref):
    nf = (1.0 - MOM) * gfm_ref[...] + MOM * feat_ref[...]
    n1 = (1.0 - MOM) * gm1_ref[...] + MOM * s1_ref[...]
    n2 = (1.0 - MOM) * gm2_ref[...] + MOM * s2_ref[...]
    nf_ref[...] = nf
    vbias = vb_ref[0:1, :]

    def corr(q, nr, out_ref):
        rn = jnp.sqrt(jnp.sum(nr * nr, axis=1))
        rinv = 1.0 / jnp.maximum(rn, EPS)
        cm = jax.lax.dot_general(q, nr, (((1,), (1,)), ((), ())),
                                 preferred_element_type=jnp.float32)
        out_ref[...] = cm * rinv[None, :] + vbias

    corr(qf_ref[...], nf, cf_ref)
    corr(q1_ref[...], n1, c1_ref)
    corr(q2_ref[...], n2, c2_ref)
    rdu = jnp.sum(jnp.abs(n1 - n2), axis=1)
    rdu_ref[...] = jnp.broadcast_to(rdu[None, :], (8, B))


@jax.jit
def _corr(feat, s1, s2, qf, q1, q2, gfm, gm1, gm2, vbias2d):
    out_shapes = (
        jax.ShapeDtypeStruct((B, B), jnp.float32),
        jax.ShapeDtypeStruct((B, B), jnp.float32),
        jax.ShapeDtypeStruct((B, B), jnp.float32),
        jax.ShapeDtypeStruct((B, D), jnp.float32),
        jax.ShapeDtypeStruct((8, B), jnp.float32),
    )
    return pl.pallas_call(_corr_kernel, out_shape=out_shapes)(
        feat, s1, s2, qf, q1, q2, gfm, gm1, gm2, vbias2d)


def _sel_kernel(cand_ref, corr_ref, pos_ref, w_ref):
    for k in range(9):
        w_ref[:, k * TBLK:(k + 1) * TBLK] = cand_ref[k]
    w_ref[:, NCAND:] = corr_ref[...]
    iota = jax.lax.broadcasted_iota(jnp.int32, (B, WTOT), 1)
    cols = []
    for _ in range(9):
        w = w_ref[...]
        m = jnp.max(w, axis=1)
        hit = w >= m[:, None]
        pos = jnp.min(jnp.where(hit, iota, BIGI), axis=1)
        cols.append(pos[:, None])
        w_ref[...] = jnp.where(hit, NEG, w)
    cols.append(jnp.zeros((B, NSLOT - 9), jnp.int32))
    pos_ref[...] = jnp.concatenate(cols, axis=1)


@jax.jit
def _sel(cand3d, corrm):
    return pl.pallas_call(
        _sel_kernel,
        out_shape=jax.ShapeDtypeStruct((B, NSLOT), jnp.int32),
        scratch_shapes=[pltpu.VMEM((B, WTOT), jnp.float32)],
    )(cand3d, corrm)


NW = 32
NROWS = 9 * B
ROWS_PER_W = NROWS // NW
SC_CHUNK = 96


@functools.partial(
    pl.kernel,
    mesh=plsc.VectorSubcoreMesh(core_axis_name="c", subcore_axis_name="s"),
    out_type=[jax.ShapeDtypeStruct((NROWS, TBLK), jnp.float32)] * 3,
    scratch_types=[
        pltpu.VMEM((SC_CHUNK,), jnp.int32),
        pltpu.VMEM((SC_CHUNK, TBLK), jnp.float32),
        pltpu.SemaphoreType.DMA,
    ],
)
def _sc_cand_gather(sf, s1, s2, jf, j1, j2, of, o1, o2, idx_v, rows_v, sem):
    wid = lax.axis_index("s") * 2 + lax.axis_index("c")
    for tab, ind, out in ((sf, jf, of), (s1, j1, o1), (s2, j2, o2)):
        for c in range(ROWS_PER_W // SC_CHUNK):
            base = wid * ROWS_PER_W + c * SC_CHUNK
            pltpu.sync_copy(ind.at[pl.ds(base, SC_CHUNK)], idx_v)
            pltpu.async_copy(tab.at[idx_v], rows_v, sem).wait()
            pltpu.sync_copy(rows_v, out.at[pl.ds(base, SC_CHUNK)])


def _feat_loss_kernel(a_ref, b_ref, out_ref, acc_ref):
    i = pl.program_id(0)

    @pl.when(i == 0)
    def _():
        acc_ref[0] = 0.0

    a = a_ref[...]
    b = b_ref[...]
    ra = jnp.maximum(jnp.sqrt(jnp.sum(a * a, axis=1)), EPS)
    rb = jnp.maximum(jnp.sqrt(jnp.sum(b * b, axis=1)), EPS)
    dots = jnp.sum(a * b, axis=1) / (ra * rb)
    acc_ref[0] += jnp.sum(dots)

    @pl.when(i == pl.num_programs(0) - 1)
    def _():
        out_ref[0] = acc_ref[0]


@jax.jit
def _feat_loss(a, b):
    n = a.shape[0]
    blk = 1024
    grid = (n // blk,)
    return pl.pallas_call(
        _feat_loss_kernel,
        grid=grid,
        in_specs=[pl.BlockSpec((blk, D), lambda i: (i, 0))] * 2,
        out_specs=pl.BlockSpec(memory_space=pltpu.SMEM),
        out_shape=jax.ShapeDtypeStruct((1,), jnp.float32),
        scratch_shapes=[pltpu.SMEM((1,), jnp.float32)],
    )(a, b)


def kernel(target_feature, fc1, fc2, index_target, target_featurememory,
           target_softmaxFc1memory, target_softmaxFc2memory):
    feat = target_feature
    idx = index_target
    FM, M1, M2 = target_featurememory, target_softmaxFc1memory, target_softmaxFc2memory

    qf, s1, q1, s2, q2 = _prep(feat, fc1, fc2)

    jpos = jnp.arange(B, dtype=jnp.int32)
    w = jnp.full((T,), -1, dtype=jnp.int32).at[idx].max(jpos)
    valid = (w[idx] == jpos)
    vbias2d = jnp.broadcast_to(
        jnp.where(valid, 0.0, NEG)[None, :], (8, B))

    mask1 = jnp.zeros((TPAD,), jnp.float32).at[idx].set(NEG)
    mask2d = jnp.broadcast_to(mask1[None, :], (8, TPAD))

    Sf, S1, S2, bf, b1, b2, rd2d = _big(qf, q1, q2, mask2d, FM, M1, M2)
    rowdiff = rd2d[0, :T]

    gfm, gm1, gm2 = FM[idx], M1[idx], M2[idx]
    Cf, C1, C2, nf, rdu2d = _corr(feat, s1, s2, qf, q1, q2, gfm, gm1, gm2,
                                  vbias2d)
    rowdiff_upd = rdu2d[0, :]

    blk9_f, blk9_1, blk9_2 = bf[:, :9], b1[:, :9], b2[:, :9]
    rowid = jnp.arange(B, dtype=jnp.int32)[:, None]
    jf = (blk9_f * B + rowid).T.reshape(-1)
    j1 = (blk9_1 * B + rowid).T.reshape(-1)
    j2 = (blk9_2 * B + rowid).T.reshape(-1)
    cand_f, cand_1, cand_2 = _sc_cand_gather(
        Sf.reshape(NBLK * B, TBLK), S1.reshape(NBLK * B, TBLK),
        S2.reshape(NBLK * B, TBLK), jf, j1, j2)

    def topk9(cand, blk9, Cm):
        pos = _sel(cand.reshape(9, B, TBLK), Cm)[:, :9]
        incand = pos < NCAND
        pc = jnp.minimum(pos, NCAND - 1)
        gblk = jnp.take_along_axis(blk9, pc // TBLK, axis=1)
        gc = gblk * TBLK + pc % TBLK
        gr = idx[jnp.clip(pos - NCAND, 0, B - 1)]
        return jnp.where(incand, gc, gr)

    f_idx = topk9(cand_f, blk9_f, Cf)[:, 1:9].reshape(-1)
    i1 = topk9(cand_1, blk9_1, C1)[:, 1:8].reshape(-1)
    i2 = topk9(cand_2, blk9_2, C2)[:, 1:8].reshape(-1)

    posf = w[f_idx]
    vals = jnp.where(posf >= 0, rowdiff_upd[jnp.maximum(posf, 0)], rowdiff[f_idx])
    loss_softmax = jnp.sum(vals) / (B * 8 * C)

    def fetch(ii):
        p = w[ii]
        base = FM[ii]
        upd = nf[jnp.maximum(p, 0)]
        return jnp.where((p >= 0)[:, None], upd, base)

    a = fetch(i1)
    b = fetch(i2)
    n = i1.shape[0]
    dotsum = _feat_loss(a, b)[0]
    loss_feature = 0.5 - dotsum / (2.0 * n)

    return (loss_softmax, loss_feature)
